# Initial kernel scaffold; baseline (speedup 1.0000x reference)
import jax
import jax.numpy as jnp
from jax.experimental import pallas as pl


def kernel(hn, edge_index, he, W, b):
    raise NotImplementedError("write your pallas kernel here")


# speedup vs baseline: 8.4174x; 8.4174x over previous
import functools

import jax
import jax.numpy as jnp
from jax import lax
from jax.experimental import pallas as pl
from jax.experimental.pallas import tpu as pltpu
from jax.experimental.pallas import tpu_sc as plsc

_C = 128


def _make_aggregate(n_nodes, n_pad, d, g, nc, ns):
    nw = nc * ns
    rows_per_sub = n_pad // ns
    mesh = plsc.VectorSubcoreMesh(core_axis_name="c", subcore_axis_name="s")

    @functools.partial(
        pl.kernel,
        mesh=mesh,
        out_type=jax.ShapeDtypeStruct((nc, n_pad, d), jnp.float32),
        scratch_types=[
            pltpu.VMEM((g, _C), jnp.int32),
            pltpu.VMEM((g, _C), jnp.int32),
            pltpu.VMEM((_C, d), jnp.float32),
            pltpu.VMEM_SHARED((n_pad, d), jnp.float32),
            pltpu.SemaphoreType.DMA,
        ],
    )
    def aggregate(hn_h, src_h, dst_h, zero_h, out_h, src_v, dst_v, rows_v, acc, sem):
        c = lax.axis_index("c")
        s = lax.axis_index("s")
        wid = s * nc + c
        base = s * rows_per_sub
        pltpu.sync_copy(zero_h.at[pl.ds(base, rows_per_sub)],
                        acc.at[pl.ds(base, rows_per_sub)])
        pltpu.sync_copy(src_h.at[wid], src_v)
        pltpu.sync_copy(dst_h.at[wid], dst_v)
        plsc.subcore_barrier()

        def step(i, carry):
            pltpu.async_copy(hn_h.at[src_v.at[i]], rows_v, sem).wait()
            pltpu.sync_copy(rows_v, acc.at[dst_v.at[i]], add=True)
            return carry

        lax.fori_loop(0, g, step, 0)
        plsc.subcore_barrier()
        pltpu.sync_copy(acc.at[pl.ds(base, rows_per_sub)],
                        out_h.at[c, pl.ds(base, rows_per_sub)])

    return aggregate


def _mm_body(hn_ref, p_ref, wt_ref, b_ref, o_ref, *, d):
    x = hn_ref[...]
    psum = p_ref[0] + p_ref[1]
    w1 = wt_ref[:d]
    w2 = wt_ref[d:]
    acc = jnp.dot(x, w1, preferred_element_type=jnp.float32,
                  precision=lax.Precision.HIGHEST)
    acc = acc + jnp.dot(psum, w2, preferred_element_type=jnp.float32,
                        precision=lax.Precision.HIGHEST)
    o_ref[...] = acc + b_ref[...]


def kernel(hn, edge_index, he, W, b):
    del he
    n, d = hn.shape
    d_out = W.shape[0]
    e = edge_index.shape[1]

    info = plsc.get_sparse_core_info()
    nc, ns = info.num_cores, info.num_subcores
    nw = nc * ns

    per_w = -(-e // (nw * _C)) * _C
    g = per_w // _C
    e_pad = per_w * nw
    n_pad = -(-(n + 1) // (ns * 8)) * (ns * 8)
    n_trash = n_pad - n

    src = edge_index[0].astype(jnp.int32)
    dst = edge_index[1].astype(jnp.int32)
    pad = e_pad - e
    if pad:
        fill = jnp.arange(pad, dtype=jnp.int32)
        src = jnp.concatenate([src, fill % n])
        dst = jnp.concatenate([dst, n + (fill % n_trash)])
    src3 = src.reshape(nw, g, _C)
    dst3 = dst.reshape(nw, g, _C)
    zeros = jnp.zeros((n_pad, d), jnp.float32)

    partials = _make_aggregate(n, n_pad, d, g, nc, ns)(hn, src3, dst3, zeros)

    bm = 1000
    wt = W.T
    b2 = b.reshape(1, d_out)
    out = pl.pallas_call(
        functools.partial(_mm_body, d=d),
        grid=(n // bm,),
        in_specs=[
            pl.BlockSpec((bm, d), lambda i: (i, 0)),
            pl.BlockSpec((2, bm, d), lambda i: (0, i, 0)),
            pl.BlockSpec((2 * d, d_out), lambda i: (0, 0)),
            pl.BlockSpec((1, d_out), lambda i: (0, 0)),
        ],
        out_specs=pl.BlockSpec((bm, d_out), lambda i: (i, 0)),
        out_shape=jax.ShapeDtypeStruct((n, d_out), jnp.float32),
    )(hn, partials, wt, b2)
    return out
